# baseline (speedup 1.0000x reference)
import jax
import jax.numpy as jnp
from jax.experimental import pallas as pl
from jax.experimental.pallas import tpu as pltpu

_B, _N2, _N1, _F, _A, _O, _L = 1024, 10, 25, 256, 128, 256, 50
_BB = 32
_STEPS = _B // _BB
_R = _BB * _N2 * _N1
_G = _BB * _N2
_CB = 128


def _l2n(x):
    return x * jax.lax.rsqrt(jnp.maximum(jnp.sum(x * x, axis=-1, keepdims=True), 1e-12))


def _fused(x0_ref, x1_ref, x2_ref, s1_ref, s2_ref, wagg0_ref, w0s_ref,
           w0a_ref, wagg1_ref, w1s_ref, w1a_ref, wcls_ref, out_ref, agg0_ref):
    i = pl.program_id(0)
    x2 = x2_ref[...].astype(jnp.bfloat16)
    t = jnp.maximum(jnp.dot(x2, wagg0_ref[...], preferred_element_type=jnp.float32), 0.0)
    agg0 = jnp.dot(s1_ref[...], t.astype(jnp.bfloat16),
                   preferred_element_type=jnp.float32) * (1.0 / _N1)
    agg0_ref[pl.ds(i * _G, _G), :] = agg0.astype(jnp.bfloat16)

    @pl.when(i == _STEPS - 1)
    def _tail():
        for k in range(_B // _CB):
            rows1 = pl.ds(k * _CB * _N2, _CB * _N2)
            x1c = x1_ref[rows1, :].astype(jnp.bfloat16)
            a0c = agg0_ref[rows1, :]
            h1 = jnp.maximum(
                jnp.dot(x1c, w0s_ref[...], preferred_element_type=jnp.float32)
                + jnp.dot(a0c, w0a_ref[...], preferred_element_type=jnp.float32), 0.0)
            h1 = _l2n(h1).astype(jnp.bfloat16)
            g = jnp.maximum(jnp.dot(h1, wagg1_ref[...], preferred_element_type=jnp.float32), 0.0)
            agg1 = jnp.dot(s2_ref[...], g.astype(jnp.bfloat16),
                           preferred_element_type=jnp.float32) * (1.0 / _N2)
            rows0 = pl.ds(k * _CB, _CB)
            h0 = (jnp.dot(x0_ref[rows0, :], w1s_ref[...], preferred_element_type=jnp.float32)
                  + jnp.dot(agg1, w1a_ref[...], preferred_element_type=jnp.float32))
            h0 = _l2n(_l2n(h0))
            out_ref[rows0, :] = jnp.maximum(
                jnp.dot(h0, wcls_ref[...], preferred_element_type=jnp.float32), 0.0)


def _full(shape):
    return pl.BlockSpec(shape, lambda i: (0,) * len(shape))


def kernel(x0, x1, x2, Wagg0, Wagg1, Wcomb0, Wcomb1, Wcls):
    w0s = Wcomb0[:_F].astype(jnp.bfloat16)
    w0a = Wcomb0[_F:].astype(jnp.bfloat16)
    w1s, w1a = Wcomb1[:_F], Wcomb1[_F:]
    x2r = x2.reshape(_B * _N2 * _N1, _F)
    x1r = x1.reshape(_B * _N2, _F)
    s1 = (jnp.arange(_R, dtype=jnp.int32)[None, :] // _N1
          == jnp.arange(_G, dtype=jnp.int32)[:, None]).astype(jnp.bfloat16)
    s2 = (jnp.arange(_CB * _N2, dtype=jnp.int32)[None, :] // _N2
          == jnp.arange(_CB, dtype=jnp.int32)[:, None]).astype(jnp.bfloat16)
    return pl.pallas_call(
        _fused,
        grid=(_STEPS,),
        in_specs=[
            _full((_B, _F)),
            _full((_B * _N2, _F)),
            pl.BlockSpec((_R, _F), lambda i: (i, 0)),
            _full((_G, _R)),
            _full((_CB, _CB * _N2)),
            _full((_F, _A)), _full((_F, _O)), _full((_A, _O)),
            _full((_O, _A)), _full((_F, _O)), _full((_A, _O)),
            _full((_O, _L)),
        ],
        out_specs=_full((_B, _L)),
        out_shape=jax.ShapeDtypeStruct((_B, _L), jnp.float32),
        scratch_shapes=[pltpu.VMEM((_B * _N2, _A), jnp.bfloat16)],
        compiler_params=pltpu.CompilerParams(dimension_semantics=("arbitrary",)),
    )(x0, x1r, x2r, s1, s2, Wagg0.astype(jnp.bfloat16), w0s, w0a,
      Wagg1.astype(jnp.bfloat16), w1s, w1a, Wcls)


# speedup vs baseline: 1.7435x; 1.7435x over previous
import jax
import jax.numpy as jnp
from jax.experimental import pallas as pl
from jax.experimental.pallas import tpu as pltpu

_B, _N2, _N1, _F, _A, _O, _L = 1024, 10, 25, 256, 128, 256, 50
_N1P, _N2P = 32, 16
_BB = 32
_STEPS = _B // _BB
_G = _BB * _N2
_CB = 64


def _l2n(x):
    return x * jax.lax.rsqrt(jnp.maximum(jnp.sum(x * x, axis=-1, keepdims=True), 1e-12))


def _fused(x0_ref, x1_ref, x2_ref, wagg0_ref, w0s_ref, w0a_ref, wagg1_ref,
           w1s_ref, w1a_ref, wcls_ref, out_ref, agg0_ref):
    i = pl.program_id(0)
    x2 = jnp.pad(x2_ref[...], ((0, 0), (0, 0), (0, _N1P - _N1), (0, 0)))
    x2 = x2.reshape(_BB * _N2 * _N1P, _F).astype(jnp.bfloat16)
    t = jnp.maximum(jnp.dot(x2, wagg0_ref[...], preferred_element_type=jnp.float32), 0.0)
    agg0 = jnp.sum(t.reshape(_BB, _N2, _N1P, _A), axis=2) * (1.0 / _N1)
    agg0 = jnp.pad(agg0, ((0, 0), (0, _N2P - _N2), (0, 0)))
    agg0_ref[pl.ds(i * _BB * _N2P, _BB * _N2P), :] = (
        agg0.reshape(_BB * _N2P, _A).astype(jnp.bfloat16))

    @pl.when(i % 2 == 1)
    def _tail():
        k = i // 2
        x1 = jnp.pad(x1_ref[pl.ds(k * _CB, _CB)], ((0, 0), (0, _N2P - _N2), (0, 0)))
        x1 = x1.reshape(_CB * _N2P, _F).astype(jnp.bfloat16)
        a0 = agg0_ref[pl.ds(k * _CB * _N2P, _CB * _N2P), :]
        h1 = jnp.maximum(
            jnp.dot(x1, w0s_ref[...], preferred_element_type=jnp.float32)
            + jnp.dot(a0, w0a_ref[...], preferred_element_type=jnp.float32), 0.0)
        h1 = _l2n(h1).astype(jnp.bfloat16)
        g = jnp.maximum(jnp.dot(h1, wagg1_ref[...], preferred_element_type=jnp.float32), 0.0)
        agg1 = jnp.sum(g.reshape(_CB, _N2P, _A), axis=1) * (1.0 / _N2)
        rows0 = pl.ds(k * _CB, _CB)
        h0 = (jnp.dot(x0_ref[rows0, :], w1s_ref[...], preferred_element_type=jnp.float32)
              + jnp.dot(agg1, w1a_ref[...], preferred_element_type=jnp.float32))
        h0 = _l2n(_l2n(h0))
        out_ref[rows0, :] = jnp.maximum(
            jnp.dot(h0, wcls_ref[...], preferred_element_type=jnp.float32), 0.0)


def _full(shape):
    return pl.BlockSpec(shape, lambda i: (0,) * len(shape))


def kernel(x0, x1, x2, Wagg0, Wagg1, Wcomb0, Wcomb1, Wcls):
    w0s = Wcomb0[:_F].astype(jnp.bfloat16)
    w0a = Wcomb0[_F:].astype(jnp.bfloat16)
    w1s, w1a = Wcomb1[:_F], Wcomb1[_F:]
    return pl.pallas_call(
        _fused,
        grid=(_STEPS,),
        in_specs=[
            _full((_B, _F)),
            _full((_B, _N2, _F)),
            pl.BlockSpec((_BB, _N2, _N1, _F), lambda i: (i, 0, 0, 0)),
            _full((_F, _A)), _full((_F, _O)), _full((_A, _O)),
            _full((_O, _A)), _full((_F, _O)), _full((_A, _O)),
            _full((_O, _L)),
        ],
        out_specs=_full((_B, _L)),
        out_shape=jax.ShapeDtypeStruct((_B, _L), jnp.float32),
        scratch_shapes=[pltpu.VMEM((_B * _N2P, _A), jnp.bfloat16)],
        compiler_params=pltpu.CompilerParams(dimension_semantics=("arbitrary",)),
    )(x0, x1, x2, Wagg0.astype(jnp.bfloat16), w0s, w0a,
      Wagg1.astype(jnp.bfloat16), w1s, w1a, Wcls)


# speedup vs baseline: 1.7642x; 1.0119x over previous
import jax
import jax.numpy as jnp
from jax.experimental import pallas as pl
from jax.experimental.pallas import tpu as pltpu

_B, _N2, _N1, _F, _A, _O, _L = 1024, 10, 25, 256, 128, 256, 50
_N1P, _N2P = 32, 16
_BB = 32
_STEPS = _B // _BB
_G = _BB * _N2
_CB = 64


def _l2n(x):
    return x * jax.lax.rsqrt(jnp.maximum(jnp.sum(x * x, axis=-1, keepdims=True), 1e-12))


def _fused(x0_ref, x1_ref, x2_ref, wagg0_ref, w0s_ref, w0a_ref, wagg1_ref,
           w1s_ref, w1a_ref, wcls_ref, out_ref, agg0_ref):
    i = pl.program_id(0)
    x2 = jnp.pad(x2_ref[...], ((0, 0), (0, 0), (0, _N1P - _N1), (0, 0)))
    x2 = x2.reshape(_BB * _N2 * _N1P, _F).astype(jnp.bfloat16)
    t = jnp.maximum(jnp.dot(x2, wagg0_ref[...], preferred_element_type=jnp.float32), 0.0)
    agg0 = jnp.sum(t.reshape(_BB, _N2, _N1P, _A), axis=2) * (1.0 / _N1)
    agg0 = jnp.pad(agg0, ((0, 0), (0, _N2P - _N2), (0, 0)))
    agg0_ref[pl.ds(i * _BB * _N2P, _BB * _N2P), :] = (
        agg0.reshape(_BB * _N2P, _A).astype(jnp.bfloat16))

    @pl.when(i % 2 == 1)
    def _tail():
        k = i // 2
        x1 = jnp.pad(x1_ref[...], ((0, 0), (0, _N2P - _N2), (0, 0)))
        x1 = x1.reshape(_CB * _N2P, _F).astype(jnp.bfloat16)
        a0 = agg0_ref[pl.ds(k * _CB * _N2P, _CB * _N2P), :]
        h1 = jnp.maximum(
            jnp.dot(x1, w0s_ref[...], preferred_element_type=jnp.float32)
            + jnp.dot(a0, w0a_ref[...], preferred_element_type=jnp.float32), 0.0)
        h1 = _l2n(h1).astype(jnp.bfloat16)
        g = jnp.maximum(jnp.dot(h1, wagg1_ref[...], preferred_element_type=jnp.float32), 0.0)
        agg1 = jnp.sum(g.reshape(_CB, _N2P, _A), axis=1) * (1.0 / _N2)
        h0 = (jnp.dot(x0_ref[...], w1s_ref[...], preferred_element_type=jnp.float32)
              + jnp.dot(agg1, w1a_ref[...], preferred_element_type=jnp.float32))
        h0 = _l2n(_l2n(h0))
        out_ref[...] = jnp.maximum(
            jnp.dot(h0, wcls_ref[...], preferred_element_type=jnp.float32), 0.0)


def _full(shape):
    return pl.BlockSpec(shape, lambda i: (0,) * len(shape))


def kernel(x0, x1, x2, Wagg0, Wagg1, Wcomb0, Wcomb1, Wcls):
    w0s = Wcomb0[:_F].astype(jnp.bfloat16)
    w0a = Wcomb0[_F:].astype(jnp.bfloat16)
    w1s, w1a = Wcomb1[:_F], Wcomb1[_F:]
    return pl.pallas_call(
        _fused,
        grid=(_STEPS,),
        in_specs=[
            pl.BlockSpec((_CB, _F), lambda i: (i // 2, 0)),
            pl.BlockSpec((_CB, _N2, _F), lambda i: (i // 2, 0, 0)),
            pl.BlockSpec((_BB, _N2, _N1, _F), lambda i: (i, 0, 0, 0)),
            _full((_F, _A)), _full((_F, _O)), _full((_A, _O)),
            _full((_O, _A)), _full((_F, _O)), _full((_A, _O)),
            _full((_O, _L)),
        ],
        out_specs=pl.BlockSpec((_CB, _L), lambda i: (i // 2, 0)),
        out_shape=jax.ShapeDtypeStruct((_B, _L), jnp.float32),
        scratch_shapes=[pltpu.VMEM((_B * _N2P, _A), jnp.bfloat16)],
        compiler_params=pltpu.CompilerParams(dimension_semantics=("arbitrary",)),
    )(x0, x1, x2, Wagg0.astype(jnp.bfloat16), w0s, w0a,
      Wagg1.astype(jnp.bfloat16), w1s, w1a, Wcls)
